# baseline (speedup 1.0000x reference)
import jax
import jax.numpy as jnp
import numpy as np
from jax.experimental import pallas as pl
from jax.experimental.pallas import tpu as pltpu
from jax.experimental.pallas import tpu_sc as plsc

N_CODES = 8192
C_DIM = 256
N_TOK = 8192
TOK_TILE = 2048
CODE_TILE = 1024
N_CTILES = N_CODES // CODE_TILE
GATHER_WINDOW = 128


def _vq_tc_kernel(z2_ref, z_ref, e_ref, idx_ref, loss_ref, rv_ref, ri_ref,
                  zbf_ref, iof_ref, mma_ref, mmb_ref, e2a_ref, e2b_ref):
    i = pl.program_id(0)
    j = pl.program_id(1)

    @pl.when((i == 0) & (j == 0))
    def _once():
        iof_ref[...] = jax.lax.broadcasted_iota(
            jnp.int32, iof_ref.shape, 0
        ).astype(jnp.float32)

    def _matmul(mm_ref, e2_ref):
        e = e_ref[...]
        e2x = (e + e).astype(jnp.bfloat16)
        mm_ref[...] = jax.lax.dot_general(
            e2x, zbf_ref[...], (((1,), (1,)), ((), ())),
            preferred_element_type=jnp.float32,
        )
        e2_ref[...] = jnp.sum(e * e, axis=1, keepdims=True)

    def _reduce(mm_ref, e2_ref):
        d = (z2_ref[0] - mm_ref[...]) + e2_ref[...]
        cmin = jnp.min(d, axis=0, keepdims=True)
        local = jnp.min(
            jnp.where(d == cmin, iof_ref[...], jnp.float32(CODE_TILE)),
            axis=0, keepdims=True,
        )
        cidx = local + ((j - 1) * CODE_TILE).astype(jnp.float32)
        better = cmin < rv_ref[...]
        rv_ref[...] = jnp.where(better, cmin, rv_ref[...])
        ri_ref[...] = jnp.where(better, cidx, ri_ref[...])

    @pl.when(j == 0)
    def _first():
        rv_ref[...] = jnp.full_like(rv_ref[...], jnp.inf)
        ri_ref[...] = jnp.zeros_like(ri_ref[...])
        zbf_ref[...] = z_ref[...].astype(jnp.bfloat16)
        _matmul(mma_ref, e2a_ref)

    @pl.when((j % 2) == 1)
    def _odd():
        _matmul(mmb_ref, e2b_ref)
        _reduce(mma_ref, e2a_ref)

    @pl.when(((j % 2) == 0) & (j > 0))
    def _even():
        _matmul(mma_ref, e2a_ref)
        _reduce(mmb_ref, e2b_ref)

    @pl.when(j == pl.num_programs(1) - 1)
    def _finish():
        idx_ref[0] = ri_ref[...].astype(jnp.int32)
        loss_ref[...] = jnp.sum(rv_ref[...]).reshape(1, 1, 1)


def _vq_argmin(z2, z_flat, embedding_weight):
    n_tiles = N_TOK // TOK_TILE
    grid = (n_tiles, N_CTILES + 1)
    return pl.pallas_call(
        _vq_tc_kernel,
        grid=grid,
        in_specs=[
            pl.BlockSpec((1, 1, TOK_TILE), lambda i, j: (i, 0, 0)),
            pl.BlockSpec((TOK_TILE, C_DIM), lambda i, j: (i, 0)),
            pl.BlockSpec(
                (CODE_TILE, C_DIM),
                lambda i, j: (jnp.minimum(j, N_CTILES - 1), 0),
            ),
        ],
        out_specs=[
            pl.BlockSpec((1, 1, TOK_TILE), lambda i, j: (i, 0, 0)),
            pl.BlockSpec((1, 1, 1), lambda i, j: (i, 0, 0)),
        ],
        out_shape=[
            jax.ShapeDtypeStruct((n_tiles, 1, TOK_TILE), jnp.int32),
            jax.ShapeDtypeStruct((n_tiles, 1, 1), jnp.float32),
        ],
        scratch_shapes=[
            pltpu.VMEM((1, TOK_TILE), jnp.float32),
            pltpu.VMEM((1, TOK_TILE), jnp.float32),
            pltpu.VMEM((TOK_TILE, C_DIM), jnp.bfloat16),
            pltpu.VMEM((CODE_TILE, 1), jnp.float32),
            pltpu.VMEM((CODE_TILE, TOK_TILE), jnp.float32),
            pltpu.VMEM((CODE_TILE, TOK_TILE), jnp.float32),
            pltpu.VMEM((CODE_TILE, 1), jnp.float32),
            pltpu.VMEM((CODE_TILE, 1), jnp.float32),
        ],
        compiler_params=pltpu.CompilerParams(
            dimension_semantics=("parallel", "arbitrary"),
        ),
    )(z2.reshape(n_tiles, 1, TOK_TILE), z_flat, embedding_weight)


def _sc_gather(e_hbm, indices):
    mesh = plsc.VectorSubcoreMesh(
        core_axis_name="core", subcore_axis_name="subcore"
    )
    idx2 = indices.reshape(1, N_TOK)

    @pl.kernel(
        out_type=jax.ShapeDtypeStruct((N_TOK, C_DIM), e_hbm.dtype), mesh=mesh
    )
    def k(e_ref, i_ref, o_ref):
        def body(i_vmem, o_vmem):
            pltpu.sync_copy(e_ref.at[i_vmem.at[0]], o_vmem)

        pltpu.emit_pipeline(
            body,
            grid=(N_TOK // GATHER_WINDOW,),
            in_specs=[
                pl.BlockSpec((1, GATHER_WINDOW), index_map=lambda s: (0, s))
            ],
            out_specs=[
                pl.BlockSpec(
                    (GATHER_WINDOW, C_DIM), index_map=lambda s: (s, 0)
                )
            ],
            core_axis_name=("core", "subcore"),
            dimension_semantics=(pltpu.PARALLEL,),
        )(i_ref, o_ref)

    return k(e_hbm, idx2)


def kernel(z, embedding_weight):
    B, D, H, W = z.shape
    z_flat = jnp.transpose(z, (0, 2, 3, 1)).reshape(-1, D)
    z2 = jnp.sum(z_flat ** 2, axis=1)
    idx3, loss_parts = _vq_argmin(z2, z_flat, embedding_weight)
    indices = idx3.reshape(N_TOK)
    zq_flat = _sc_gather(embedding_weight, indices)
    z_q = jnp.transpose(zq_flat.reshape(B, H, W, D), (0, 3, 1, 2))
    loss = jnp.sum(loss_parts) * np.float32(2.0 ** -23)
    return (z_q, idx3.reshape(B, H, W), loss)


# speedup vs baseline: 1.3160x; 1.3160x over previous
import jax
import jax.numpy as jnp
import numpy as np
from jax.experimental import pallas as pl
from jax.experimental.pallas import tpu as pltpu
from jax.experimental.pallas import tpu_sc as plsc

N_CODES = 8192
C_DIM = 256
N_TOK = 8192
TOK_TILE = 2048
CODE_TILE = 2048
GATHER_WINDOW = 128


def _vq_tc_kernel(z2_ref, z_ref, e_ref, idx_ref, loss_ref, rv_ref, ri_ref,
                  zbf_ref, iof_ref, e2xs_ref, e2s_ref):
    i = pl.program_id(0)
    j = pl.program_id(1)

    @pl.when((i == 0) & (j == 0))
    def _once():
        iof_ref[...] = jax.lax.broadcasted_iota(
            jnp.int32, iof_ref.shape, 0
        ).astype(jnp.float32)

    @pl.when(j == 0)
    def _init():
        rv_ref[...] = jnp.full_like(rv_ref[...], jnp.inf)
        ri_ref[...] = jnp.zeros_like(ri_ref[...])
        zbf_ref[...] = z_ref[...].astype(jnp.bfloat16)

    sl = pl.ds(j * CODE_TILE, CODE_TILE)

    @pl.when(i == 0)
    def _prep_codes():
        e = e_ref[...]
        e2xs_ref[sl, :] = (e + e).astype(jnp.bfloat16)
        e2s_ref[sl, :] = jnp.sum(e * e, axis=1, keepdims=True)

    mm2 = jax.lax.dot_general(
        e2xs_ref[sl, :], zbf_ref[...], (((1,), (1,)), ((), ())),
        preferred_element_type=jnp.float32,
    )
    e2 = e2s_ref[sl, :]
    z2 = z2_ref[0]
    d = (z2 - mm2) + e2

    cmin = jnp.min(d, axis=0, keepdims=True)
    local = jnp.min(
        jnp.where(d == cmin, iof_ref[...], jnp.float32(CODE_TILE)),
        axis=0, keepdims=True,
    )
    cidx = local + (j * CODE_TILE).astype(jnp.float32)
    better = cmin < rv_ref[...]
    rv_ref[...] = jnp.where(better, cmin, rv_ref[...])
    ri_ref[...] = jnp.where(better, cidx, ri_ref[...])

    @pl.when(j == pl.num_programs(1) - 1)
    def _finish():
        idx_ref[0] = ri_ref[...].astype(jnp.int32)
        loss_ref[...] = jnp.sum(rv_ref[...]).reshape(1, 1, 1)


def _vq_argmin(z2, z_flat, embedding_weight):
    n_tiles = N_TOK // TOK_TILE
    grid = (n_tiles, N_CODES // CODE_TILE)
    return pl.pallas_call(
        _vq_tc_kernel,
        grid=grid,
        in_specs=[
            pl.BlockSpec((1, 1, TOK_TILE), lambda i, j: (i, 0, 0)),
            pl.BlockSpec((TOK_TILE, C_DIM), lambda i, j: (i, 0)),
            pl.BlockSpec((CODE_TILE, C_DIM), lambda i, j: (j, 0)),
        ],
        out_specs=[
            pl.BlockSpec((1, 1, TOK_TILE), lambda i, j: (i, 0, 0)),
            pl.BlockSpec((1, 1, 1), lambda i, j: (i, 0, 0)),
        ],
        out_shape=[
            jax.ShapeDtypeStruct((n_tiles, 1, TOK_TILE), jnp.int32),
            jax.ShapeDtypeStruct((n_tiles, 1, 1), jnp.float32),
        ],
        scratch_shapes=[
            pltpu.VMEM((1, TOK_TILE), jnp.float32),
            pltpu.VMEM((1, TOK_TILE), jnp.float32),
            pltpu.VMEM((TOK_TILE, C_DIM), jnp.bfloat16),
            pltpu.VMEM((CODE_TILE, 1), jnp.float32),
            pltpu.VMEM((N_CODES, C_DIM), jnp.bfloat16),
            pltpu.VMEM((N_CODES, 1), jnp.float32),
        ],
        compiler_params=pltpu.CompilerParams(
            dimension_semantics=("parallel", "arbitrary"),
        ),
    )(z2.reshape(n_tiles, 1, TOK_TILE), z_flat, embedding_weight)


def _sc_gather(e_hbm, indices):
    mesh = plsc.VectorSubcoreMesh(
        core_axis_name="core", subcore_axis_name="subcore"
    )
    idx2 = indices.reshape(1, N_TOK)

    @pl.kernel(
        out_type=jax.ShapeDtypeStruct((N_TOK, C_DIM), e_hbm.dtype), mesh=mesh
    )
    def k(e_ref, i_ref, o_ref):
        def body(i_vmem, o_vmem):
            pltpu.sync_copy(e_ref.at[i_vmem.at[0]], o_vmem)

        pltpu.emit_pipeline(
            body,
            grid=(N_TOK // GATHER_WINDOW,),
            in_specs=[
                pl.BlockSpec((1, GATHER_WINDOW), index_map=lambda s: (0, s))
            ],
            out_specs=[
                pl.BlockSpec(
                    (GATHER_WINDOW, C_DIM), index_map=lambda s: (s, 0)
                )
            ],
            core_axis_name=("core", "subcore"),
            dimension_semantics=(pltpu.PARALLEL,),
        )(i_ref, o_ref)

    return k(e_hbm, idx2)


def kernel(z, embedding_weight):
    B, D, H, W = z.shape
    z_flat = jnp.transpose(z, (0, 2, 3, 1)).reshape(-1, D)
    z2 = jnp.sum(z_flat ** 2, axis=1)
    idx3, loss_parts = _vq_argmin(z2, z_flat, embedding_weight)
    indices = idx3.reshape(N_TOK)
    zq_flat = _sc_gather(embedding_weight, indices)
    z_q = jnp.transpose(zq_flat.reshape(B, H, W, D), (0, 3, 1, 2))
    loss = jnp.sum(loss_parts) * np.float32(2.0 ** -23)
    return (z_q, idx3.reshape(B, H, W), loss)


# speedup vs baseline: 1.3554x; 1.0300x over previous
import jax
import jax.numpy as jnp
import numpy as np
from jax.experimental import pallas as pl
from jax.experimental.pallas import tpu as pltpu
from jax.experimental.pallas import tpu_sc as plsc

N_CODES = 8192
C_DIM = 256
N_TOK = 8192
TOK_TILE = 2048
CODE_TILE = 2048
GATHER_WINDOW = 128


def _vq_tc_kernel(z_ref, e_ref, idx_ref, loss_ref, rv_ref, ri_ref,
                  zbf_ref, iof_ref, e2xs_ref, e2s_ref, z2r_ref):
    i = pl.program_id(0)
    j = pl.program_id(1)

    @pl.when((i == 0) & (j == 0))
    def _once():
        iof_ref[...] = jax.lax.broadcasted_iota(
            jnp.int32, iof_ref.shape, 0
        ).astype(jnp.float32)

    @pl.when(j == 0)
    def _init():
        rv_ref[...] = jnp.full_like(rv_ref[...], jnp.inf)
        ri_ref[...] = jnp.zeros_like(ri_ref[...])
        zf = z_ref[...]
        zbf_ref[...] = zf.astype(jnp.bfloat16)
        z2r_ref[...] = jnp.sum(zf * zf, axis=1, keepdims=True).T

    sl = pl.ds(j * CODE_TILE, CODE_TILE)

    @pl.when(i == 0)
    def _prep_codes():
        e = e_ref[...]
        e2xs_ref[sl, :] = (e + e).astype(jnp.bfloat16)
        e2s_ref[sl, :] = jnp.sum(e * e, axis=1, keepdims=True)

    mm2 = jax.lax.dot_general(
        e2xs_ref[sl, :], zbf_ref[...], (((1,), (1,)), ((), ())),
        preferred_element_type=jnp.float32,
    )
    e2 = e2s_ref[sl, :]
    z2 = z2r_ref[...]
    d = (z2 - mm2) + e2

    cmin = jnp.min(d, axis=0, keepdims=True)
    local = jnp.min(
        jnp.where(d == cmin, iof_ref[...], jnp.float32(CODE_TILE)),
        axis=0, keepdims=True,
    )
    cidx = local + (j * CODE_TILE).astype(jnp.float32)
    better = cmin < rv_ref[...]
    rv_ref[...] = jnp.where(better, cmin, rv_ref[...])
    ri_ref[...] = jnp.where(better, cidx, ri_ref[...])

    @pl.when(j == pl.num_programs(1) - 1)
    def _finish():
        idx_ref[0] = ri_ref[...].astype(jnp.int32)
        loss_ref[...] = jnp.sum(rv_ref[...]).reshape(1, 1, 1)


def _vq_argmin(z_flat, embedding_weight):
    n_tiles = N_TOK // TOK_TILE
    grid = (n_tiles, N_CODES // CODE_TILE)
    return pl.pallas_call(
        _vq_tc_kernel,
        grid=grid,
        in_specs=[
            pl.BlockSpec((TOK_TILE, C_DIM), lambda i, j: (i, 0)),
            pl.BlockSpec((CODE_TILE, C_DIM), lambda i, j: (j, 0)),
        ],
        out_specs=[
            pl.BlockSpec((1, 1, TOK_TILE), lambda i, j: (i, 0, 0)),
            pl.BlockSpec((1, 1, 1), lambda i, j: (i, 0, 0)),
        ],
        out_shape=[
            jax.ShapeDtypeStruct((n_tiles, 1, TOK_TILE), jnp.int32),
            jax.ShapeDtypeStruct((n_tiles, 1, 1), jnp.float32),
        ],
        scratch_shapes=[
            pltpu.VMEM((1, TOK_TILE), jnp.float32),
            pltpu.VMEM((1, TOK_TILE), jnp.float32),
            pltpu.VMEM((TOK_TILE, C_DIM), jnp.bfloat16),
            pltpu.VMEM((CODE_TILE, 1), jnp.float32),
            pltpu.VMEM((N_CODES, C_DIM), jnp.bfloat16),
            pltpu.VMEM((N_CODES, 1), jnp.float32),
            pltpu.VMEM((1, TOK_TILE), jnp.float32),
        ],
        compiler_params=pltpu.CompilerParams(
            dimension_semantics=("parallel", "arbitrary"),
        ),
    )(z_flat, embedding_weight)


def _sc_gather(e_hbm, indices):
    mesh = plsc.VectorSubcoreMesh(
        core_axis_name="core", subcore_axis_name="subcore"
    )
    idx2 = indices.reshape(1, N_TOK)

    @pl.kernel(
        out_type=jax.ShapeDtypeStruct((N_TOK, C_DIM), e_hbm.dtype), mesh=mesh
    )
    def k(e_ref, i_ref, o_ref):
        def body(i_vmem, o_vmem):
            pltpu.sync_copy(e_ref.at[i_vmem.at[0]], o_vmem)

        pltpu.emit_pipeline(
            body,
            grid=(N_TOK // GATHER_WINDOW,),
            in_specs=[
                pl.BlockSpec((1, GATHER_WINDOW), index_map=lambda s: (0, s))
            ],
            out_specs=[
                pl.BlockSpec(
                    (GATHER_WINDOW, C_DIM), index_map=lambda s: (s, 0)
                )
            ],
            core_axis_name=("core", "subcore"),
            dimension_semantics=(pltpu.PARALLEL,),
        )(i_ref, o_ref)

    return k(e_hbm, idx2)


def kernel(z, embedding_weight):
    B, D, H, W = z.shape
    z_flat = jnp.transpose(z, (0, 2, 3, 1)).reshape(-1, D)
    idx3, loss_parts = _vq_argmin(z_flat, embedding_weight)
    indices = idx3.reshape(N_TOK)
    zq_flat = _sc_gather(embedding_weight, indices)
    z_q = jnp.transpose(zq_flat.reshape(B, H, W, D), (0, 3, 1, 2))
    loss = jnp.sum(loss_parts) * np.float32(2.0 ** -23)
    return (z_q, idx3.reshape(B, H, W), loss)
